# baseline (speedup 1.0000x reference)
import functools

import jax
import jax.numpy as jnp
from jax import lax
from jax.experimental import pallas as pl
from jax.experimental.pallas import tpu as pltpu
from jax.experimental.pallas import tpu_sc as plsc

N = 10000
E = 320000
D = 128

NC = 2
NS = 16
NP = 10240
RPT = NP // NS
C = 64
E_PAD = 321536
NCHUNK = E_PAD // C
NCH_PAIR = NCHUNK // NS
NCH_M0 = 134
NCH_M1 = NCH_PAIR - NCH_M0
NCH_D = NCHUNK // NS
NSLOT = 4


def _matmul_body(x_ref, w_ref, o_ref):
    o_ref[...] = jnp.dot(x_ref[...], w_ref[...],
                         preferred_element_type=jnp.float32)


def _matmul(x, W):
    return pl.pallas_call(
        _matmul_body,
        out_shape=jax.ShapeDtypeStruct((N, D), jnp.float32),
    )(x, W)


def _combine_body(a0_ref, a1_ref, d_ref, h_ref, b_ref, o_ref):
    d = d_ref[...]
    o_ref[...] = jnp.maximum(
        a0_ref[...] + a1_ref[...] + (2.0 * d * d) * h_ref[...] + b_ref[...],
        0.0)


def _combine(a0, a1, dis, h, b):
    blk = 2000
    return pl.pallas_call(
        _combine_body,
        grid=(N // blk,),
        in_specs=[
            pl.BlockSpec((blk, D), lambda i: (i, 0)),
            pl.BlockSpec((blk, D), lambda i: (i, 0)),
            pl.BlockSpec((blk, 1), lambda i: (i, 0)),
            pl.BlockSpec((blk, D), lambda i: (i, 0)),
            pl.BlockSpec((1, D), lambda i: (0, 0)),
        ],
        out_specs=pl.BlockSpec((blk, D), lambda i: (i, 0)),
        out_shape=jax.ShapeDtypeStruct((N, D), jnp.float32),
    )(a0, a1, dis.reshape(N, 1), h, b.reshape(1, D))


def _sc_body(pk_hbm, ew_hbm, h_hbm, out_hbm, dis_hbm,
             pk_v, ew_v, norm_v, rows_v, dis_full_v, deg_loc_v,
             acc_s, deg_s, dis_s, sem_ld, sem_g, sem_st):
    cid = lax.axis_index("c")
    sid = lax.axis_index("s")

    zero16 = jnp.zeros((16,), jnp.float32)
    two16 = jnp.full((16,), 2.0, jnp.float32)

    def issue_ld(ch, s):
        pltpu.async_copy(pk_hbm.at[ch], pk_v.at[s], sem_ld.at[s])
        pltpu.async_copy(ew_hbm.at[pl.ds(ch * C, C)], ew_v.at[s],
                         sem_ld.at[s])

    def wait_ld(ch, s):
        pltpu.make_async_copy(pk_hbm.at[ch], pk_v.at[s], sem_ld.at[s]).wait()
        pltpu.make_async_copy(ew_hbm.at[pl.ds(ch * C, C)], ew_v.at[s],
                              sem_ld.at[s]).wait()

    @pl.loop(0, C)
    def _(i):
        for j in range(D // 16):
            rows_v[0, i, pl.ds(j * 16, 16)] = zero16

    @pl.loop(0, RPT // 16)
    def _(k):
        deg_loc_v[pl.ds(k * 16, 16)] = two16

    pltpu.sync_copy(deg_loc_v, deg_s.at[pl.ds(sid * RPT, RPT)])

    @pl.loop(0, RPT // C)
    def _(k):
        pltpu.sync_copy(rows_v.at[0], acc_s.at[pl.ds(sid * RPT + k * C, C)])

    plsc.subcore_barrier()

    dbase = sid * NCH_D
    for s in range(NSLOT):
        issue_ld(dbase + s, s)

    @pl.loop(0, (NCH_D + 3) // NSLOT + 1)
    def _(o):
        for s in range(NSLOT):
            k = o * NSLOT + s

            @pl.when(k < NCH_D)
            def _():
                wait_ld(dbase + k, s)
                pltpu.async_copy(ew_v.at[s], deg_s.at[pk_v.at[s, 1]],
                                 sem_st.at[s], add=True)

            @pl.when(jnp.logical_and(k >= 2, k + 2 < NCH_D))
            def _():
                s2 = (s + 2) % NSLOT
                pltpu.make_async_copy(ew_v.at[s2],
                                      deg_s.at[pk_v.at[s2, 1]],
                                      sem_st.at[s2]).wait()
                issue_ld(dbase + k + 2, s2)

    for s in range(NSLOT):
        pltpu.make_async_copy(ew_v.at[s], deg_s.at[pk_v.at[s, 1]],
                              sem_st.at[s]).wait()

    plsc.subcore_barrier()

    base = sid * RPT
    pltpu.sync_copy(deg_s.at[pl.ds(base, RPT)], deg_loc_v)

    @pl.loop(0, RPT // 16)
    def _(k):
        xv = deg_loc_v[pl.ds(k * 16, 16)]
        ii = lax.bitcast_convert_type(xv, jnp.int32)
        ii = jnp.int32(0x5F3759DF) - lax.shift_right_logical(ii, 1)
        y = lax.bitcast_convert_type(ii, jnp.float32)
        for _ in range(3):
            y = y * (1.5 - 0.5 * xv * y * y)
        deg_loc_v[pl.ds(k * 16, 16)] = y

    pltpu.sync_copy(deg_loc_v, dis_s.at[pl.ds(base, RPT)])

    @pl.when(cid == 0)
    def _():
        pltpu.sync_copy(deg_loc_v, dis_hbm.at[pl.ds(base, RPT)])

    plsc.subcore_barrier()
    pltpu.sync_copy(dis_s, dis_full_v)

    nch_m = jnp.where(cid == 0, NCH_M0, NCH_M1)
    mbase = (cid * NS * NCH_M0
             + sid * nch_m)
    for s in range(NSLOT):
        issue_ld(mbase + s, s)

    @pl.loop(0, (max(NCH_M0, NCH_M1) + 1 + 3) // NSLOT + 1)
    def _(o):
        for s in range(NSLOT):
            k = o * NSLOT + s
            p = (s + 3) % NSLOT
            s2 = (s + 2) % NSLOT

            @pl.when(k < nch_m)
            def _():
                wait_ld(mbase + k, s)
                pltpu.async_copy(h_hbm.at[pk_v.at[s, 0]], rows_v.at[s],
                                 sem_g.at[s])

            @pl.when(jnp.logical_and(k >= 1, k - 1 < nch_m))
            def _():
                pltpu.make_async_copy(h_hbm.at[pk_v.at[p, 0]], rows_v.at[p],
                                      sem_g.at[p]).wait()
                for j in range(C // 16):
                    ir = pk_v[p, 0, pl.ds(j * 16, 16)]
                    ic = pk_v[p, 1, pl.ds(j * 16, 16)]
                    dr = plsc.load_gather(dis_full_v, [ir])
                    dc = plsc.load_gather(dis_full_v, [ic])
                    norm_v[pl.ds(j * 16, 16)] = (
                        dr * ew_v[p, pl.ds(j * 16, 16)] * dc)

                @pl.loop(0, C)
                def _(i):
                    nb = plsc.load_gather(norm_v,
                                          [jnp.full((16,), i, jnp.int32)])
                    for j in range(D // 16):
                        rows_v[p, i, pl.ds(j * 16, 16)] = (
                            rows_v[p, i, pl.ds(j * 16, 16)] * nb)

                pltpu.async_copy(rows_v.at[p], acc_s.at[pk_v.at[p, 1]],
                                 sem_st.at[p], add=True)

            @pl.when(jnp.logical_and(k >= 2, k + 2 < nch_m))
            def _():
                pltpu.make_async_copy(rows_v.at[s2],
                                      acc_s.at[pk_v.at[s2, 1]],
                                      sem_st.at[s2]).wait()
                issue_ld(mbase + k + 2, s2)

    for s in range(NSLOT):
        pltpu.make_async_copy(rows_v.at[s], acc_s.at[pk_v.at[s, 1]],
                              sem_st.at[s]).wait()

    plsc.subcore_barrier()

    @pl.loop(0, RPT // C)
    def _(k):
        o = sid * RPT + k * C
        pltpu.sync_copy(acc_s.at[pl.ds(o, C)], out_hbm.at[cid, pl.ds(o, C)])


_sc_agg = functools.partial(
    pl.kernel,
    out_type=(
        jax.ShapeDtypeStruct((NC, NP, D), jnp.float32),
        jax.ShapeDtypeStruct((NP,), jnp.float32),
    ),
    mesh=plsc.VectorSubcoreMesh(core_axis_name="c", subcore_axis_name="s"),
    compiler_params=pltpu.CompilerParams(needs_layout_passes=False),
    scratch_types=(
        pltpu.VMEM((NSLOT, 2, C), jnp.int32),
        pltpu.VMEM((NSLOT, C), jnp.float32),
        pltpu.VMEM((C,), jnp.float32),
        pltpu.VMEM((NSLOT, C, D), jnp.float32),
        pltpu.VMEM((NP,), jnp.float32),
        pltpu.VMEM((RPT,), jnp.float32),
        pltpu.VMEM_SHARED((NP, D), jnp.float32),
        pltpu.VMEM_SHARED((NP,), jnp.float32),
        pltpu.VMEM_SHARED((NP,), jnp.float32),
        pltpu.SemaphoreType.DMA((NSLOT,)),
        pltpu.SemaphoreType.DMA((NSLOT,)),
        pltpu.SemaphoreType.DMA((NSLOT,)),
    ),
)(_sc_body)


def kernel(x, edge_index, edge_weight, W, b):
    row = edge_index[0].astype(jnp.int32)
    col = edge_index[1].astype(jnp.int32)
    ew = edge_weight.astype(jnp.float32)
    pad = E_PAD - E
    row_p = jnp.concatenate([row, jnp.zeros((pad,), jnp.int32)])
    col_p = jnp.concatenate([col, jnp.zeros((pad,), jnp.int32)])
    ew_p = jnp.concatenate([ew, jnp.zeros((pad,), jnp.float32)])
    pk = jnp.stack([row_p.reshape(NCHUNK, C), col_p.reshape(NCHUNK, C)],
                   axis=1)
    h = _matmul(x, W)
    acc, dis = _sc_agg(pk, ew_p, h)
    return _combine(acc[0, :N], acc[1, :N], dis[:N], h, b)


# speedup vs baseline: 1.0310x; 1.0310x over previous
import functools

import jax
import jax.numpy as jnp
from jax import lax
from jax.experimental import pallas as pl
from jax.experimental.pallas import tpu as pltpu
from jax.experimental.pallas import tpu_sc as plsc

N = 10000
E = 320000
D = 128

NC = 2
NS = 16
NP = 10240
RPT = NP // NS
C = 64
E_PAD = 321536
NCHUNK = E_PAD // C
NCH_PAIR = NCHUNK // NS
NCH_M0 = NCH_PAIR // 2
NCH_M1 = NCH_PAIR - NCH_M0
NCH_D = NCHUNK // NS
NSLOT = 4


def _matmul_body(x_ref, w_ref, o_ref):
    o_ref[...] = jnp.dot(x_ref[...], w_ref[...],
                         preferred_element_type=jnp.float32)


def _matmul(x, W):
    return pl.pallas_call(
        _matmul_body,
        out_shape=jax.ShapeDtypeStruct((N, D), jnp.float32),
    )(x, W)


def _combine_body(a0_ref, a1_ref, d_ref, h_ref, b_ref, o_ref):
    d = d_ref[...]
    o_ref[...] = jnp.maximum(
        a0_ref[...] + a1_ref[...] + (2.0 * d * d) * h_ref[...] + b_ref[...],
        0.0)


def _combine(a0, a1, dis, h, b):
    blk = 2000
    return pl.pallas_call(
        _combine_body,
        grid=(N // blk,),
        in_specs=[
            pl.BlockSpec((blk, D), lambda i: (i, 0)),
            pl.BlockSpec((blk, D), lambda i: (i, 0)),
            pl.BlockSpec((blk, 1), lambda i: (i, 0)),
            pl.BlockSpec((blk, D), lambda i: (i, 0)),
            pl.BlockSpec((1, D), lambda i: (0, 0)),
        ],
        out_specs=pl.BlockSpec((blk, D), lambda i: (i, 0)),
        out_shape=jax.ShapeDtypeStruct((N, D), jnp.float32),
    )(a0, a1, dis.reshape(N, 1), h, b.reshape(1, D))


def _sc_body(pk_hbm, ew_hbm, h_hbm, out_hbm, dis_hbm,
             pk_v, ew_v, norm_v, rows_v, dis_full_v, deg_loc_v,
             acc_s, deg_s, dis_s, sem_ld, sem_g, sem_st):
    cid = lax.axis_index("c")
    sid = lax.axis_index("s")

    zero16 = jnp.zeros((16,), jnp.float32)
    two16 = jnp.full((16,), 2.0, jnp.float32)

    def issue_ld(ch, s):
        pltpu.async_copy(pk_hbm.at[ch], pk_v.at[s], sem_ld.at[s])
        pltpu.async_copy(ew_hbm.at[pl.ds(ch * C, C)], ew_v.at[s],
                         sem_ld.at[s])

    def wait_ld(ch, s):
        pltpu.make_async_copy(pk_hbm.at[ch], pk_v.at[s], sem_ld.at[s]).wait()
        pltpu.make_async_copy(ew_hbm.at[pl.ds(ch * C, C)], ew_v.at[s],
                              sem_ld.at[s]).wait()

    @pl.loop(0, C)
    def _(i):
        for j in range(D // 16):
            rows_v[0, i, pl.ds(j * 16, 16)] = zero16

    @pl.loop(0, RPT // 16)
    def _(k):
        deg_loc_v[pl.ds(k * 16, 16)] = two16

    pltpu.sync_copy(deg_loc_v, deg_s.at[pl.ds(sid * RPT, RPT)])

    @pl.loop(0, RPT // C)
    def _(k):
        pltpu.sync_copy(rows_v.at[0], acc_s.at[pl.ds(sid * RPT + k * C, C)])

    plsc.subcore_barrier()

    @pl.loop(0, NP // 16)
    def _(k):
        dis_full_v[pl.ds(k * 16, 16)] = jnp.zeros((16,), jnp.float32)

    dbase = sid * NCH_D
    for s in range(NSLOT):
        issue_ld(dbase + s, s)

    @pl.loop(0, (NCH_D + 3) // NSLOT + 1)
    def _(o):
        for s in range(NSLOT):
            k = o * NSLOT + s

            @pl.when(k < NCH_D)
            def _():
                wait_ld(dbase + k, s)
                for j in range(C // 16):
                    ic = pk_v[s, 1, pl.ds(j * 16, 16)]
                    ewv = ew_v[s, pl.ds(j * 16, 16)]
                    plsc.addupdate_scatter(dis_full_v, [ic], ewv)

            @pl.when(jnp.logical_and(k >= 2, k + 2 < NCH_D))
            def _():
                issue_ld(dbase + k + 2, (s + 2) % NSLOT)

    @pl.loop(0, NP // (C * NSLOT))
    def _(of):
        for s in range(NSLOT):
            fb = (of * NSLOT + s) * C

            @pl.when(of > 0)
            def _():
                pltpu.make_async_copy(dis_full_v.at[pl.ds(fb, C)],
                                      deg_s.at[pk_v.at[s, 0]],
                                      sem_st.at[s]).wait()
            for j in range(C // 16):
                pk_v[s, 0, pl.ds(j * 16, 16)] = (
                    jnp.arange(16, dtype=jnp.int32) + (fb + j * 16))
            pltpu.async_copy(dis_full_v.at[pl.ds(fb, C)],
                             deg_s.at[pk_v.at[s, 0]],
                             sem_st.at[s], add=True)

    for s in range(NSLOT):
        pltpu.make_async_copy(dis_full_v.at[pl.ds(0, C)],
                              deg_s.at[pk_v.at[s, 0]], sem_st.at[s]).wait()

    plsc.subcore_barrier()

    base = sid * RPT
    pltpu.sync_copy(deg_s.at[pl.ds(base, RPT)], deg_loc_v)

    @pl.loop(0, RPT // 16)
    def _(k):
        xv = deg_loc_v[pl.ds(k * 16, 16)]
        ii = lax.bitcast_convert_type(xv, jnp.int32)
        ii = jnp.int32(0x5F3759DF) - lax.shift_right_logical(ii, 1)
        y = lax.bitcast_convert_type(ii, jnp.float32)
        for _ in range(3):
            y = y * (1.5 - 0.5 * xv * y * y)
        deg_loc_v[pl.ds(k * 16, 16)] = y

    pltpu.sync_copy(deg_loc_v, dis_s.at[pl.ds(base, RPT)])

    @pl.when(cid == 0)
    def _():
        pltpu.sync_copy(deg_loc_v, dis_hbm.at[pl.ds(base, RPT)])

    plsc.subcore_barrier()
    pltpu.sync_copy(dis_s, dis_full_v)

    nch_m = jnp.where(cid == 0, NCH_M0, NCH_M1)
    mbase = (cid * NS * NCH_M0
             + sid * nch_m)
    for s in range(NSLOT):
        issue_ld(mbase + s, s)

    @pl.loop(0, (max(NCH_M0, NCH_M1) + 1 + 3) // NSLOT + 1)
    def _(o):
        for s in range(NSLOT):
            k = o * NSLOT + s
            p = (s + 3) % NSLOT
            s2 = (s + 2) % NSLOT

            @pl.when(k < nch_m)
            def _():
                wait_ld(mbase + k, s)
                pltpu.async_copy(h_hbm.at[pk_v.at[s, 0]], rows_v.at[s],
                                 sem_g.at[s])

            @pl.when(jnp.logical_and(k >= 1, k - 1 < nch_m))
            def _():
                pltpu.make_async_copy(h_hbm.at[pk_v.at[p, 0]], rows_v.at[p],
                                      sem_g.at[p]).wait()
                for j in range(C // 16):
                    ir = pk_v[p, 0, pl.ds(j * 16, 16)]
                    ic = pk_v[p, 1, pl.ds(j * 16, 16)]
                    dr = plsc.load_gather(dis_full_v, [ir])
                    dc = plsc.load_gather(dis_full_v, [ic])
                    norm_v[pl.ds(j * 16, 16)] = (
                        dr * ew_v[p, pl.ds(j * 16, 16)] * dc)

                @pl.loop(0, C, unroll=2)
                def _(i):
                    nb = plsc.load_gather(norm_v,
                                          [jnp.full((16,), i, jnp.int32)])
                    for j in range(D // 16):
                        rows_v[p, i, pl.ds(j * 16, 16)] = (
                            rows_v[p, i, pl.ds(j * 16, 16)] * nb)

                pltpu.async_copy(rows_v.at[p], acc_s.at[pk_v.at[p, 1]],
                                 sem_st.at[p], add=True)

            @pl.when(jnp.logical_and(k >= 2, k + 2 < nch_m))
            def _():
                pltpu.make_async_copy(rows_v.at[s2],
                                      acc_s.at[pk_v.at[s2, 1]],
                                      sem_st.at[s2]).wait()
                issue_ld(mbase + k + 2, s2)

    for s in range(NSLOT):
        pltpu.make_async_copy(rows_v.at[s], acc_s.at[pk_v.at[s, 1]],
                              sem_st.at[s]).wait()

    plsc.subcore_barrier()

    @pl.loop(0, RPT // C)
    def _(k):
        o = sid * RPT + k * C
        pltpu.sync_copy(acc_s.at[pl.ds(o, C)], out_hbm.at[cid, pl.ds(o, C)])


_sc_agg = functools.partial(
    pl.kernel,
    out_type=(
        jax.ShapeDtypeStruct((NC, NP, D), jnp.float32),
        jax.ShapeDtypeStruct((NP,), jnp.float32),
    ),
    mesh=plsc.VectorSubcoreMesh(core_axis_name="c", subcore_axis_name="s"),
    compiler_params=pltpu.CompilerParams(needs_layout_passes=False),
    scratch_types=(
        pltpu.VMEM((NSLOT, 2, C), jnp.int32),
        pltpu.VMEM((NSLOT, C), jnp.float32),
        pltpu.VMEM((C,), jnp.float32),
        pltpu.VMEM((NSLOT, C, D), jnp.float32),
        pltpu.VMEM((NP,), jnp.float32),
        pltpu.VMEM((RPT,), jnp.float32),
        pltpu.VMEM_SHARED((NP, D), jnp.float32),
        pltpu.VMEM_SHARED((NP,), jnp.float32),
        pltpu.VMEM_SHARED((NP,), jnp.float32),
        pltpu.SemaphoreType.DMA((NSLOT,)),
        pltpu.SemaphoreType.DMA((NSLOT,)),
        pltpu.SemaphoreType.DMA((NSLOT,)),
    ),
)(_sc_body)


def kernel(x, edge_index, edge_weight, W, b):
    row = edge_index[0].astype(jnp.int32)
    col = edge_index[1].astype(jnp.int32)
    ew = edge_weight.astype(jnp.float32)
    pad = E_PAD - E
    row_p = jnp.concatenate([row, jnp.zeros((pad,), jnp.int32)])
    col_p = jnp.concatenate([col, jnp.zeros((pad,), jnp.int32)])
    ew_p = jnp.concatenate([ew, jnp.zeros((pad,), jnp.float32)])
    pk = jnp.stack([row_p.reshape(NCHUNK, C), col_p.reshape(NCHUNK, C)],
                   axis=1)
    h = _matmul(x, W)
    acc, dis = _sc_agg(pk, ew_p, h)
    return _combine(acc[0, :N], acc[1, :N], dis[:N], h, b)


# speedup vs baseline: 1.1452x; 1.1108x over previous
import functools

import jax
import jax.numpy as jnp
from jax import lax
from jax.experimental import pallas as pl
from jax.experimental.pallas import tpu as pltpu
from jax.experimental.pallas import tpu_sc as plsc

N = 10000
E = 320000
D = 128

NC = 2
NS = 16
NP = 10240
RPT = NP // NS
C = 64
E_PAD = 321536
NCHUNK = E_PAD // C
NCH_PAIR = NCHUNK // NS
NCH_M0 = NCH_PAIR // 2
NCH_M1 = NCH_PAIR - NCH_M0
NCH_D = NCHUNK // NS
NSLOT = 4


def _matmul_body(x_ref, w_ref, o_ref):
    o_ref[...] = jnp.dot(x_ref[...], w_ref[...],
                         preferred_element_type=jnp.float32)


def _matmul(x, W):
    return pl.pallas_call(
        _matmul_body,
        out_shape=jax.ShapeDtypeStruct((N, D), jnp.float32),
    )(x, W)


def _combine_body(a0_ref, a1_ref, d_ref, h_ref, b_ref, o_ref):
    d = d_ref[...]
    o_ref[...] = jnp.maximum(
        a0_ref[0] + a1_ref[0] + (2.0 * d * d) * h_ref[...] + b_ref[...],
        0.0)


def _combine(acc, dis, h, b):
    blk = 2000
    return pl.pallas_call(
        _combine_body,
        grid=(N // blk,),
        in_specs=[
            pl.BlockSpec((1, blk, D), lambda i: (0, i, 0)),
            pl.BlockSpec((1, blk, D), lambda i: (1, i, 0)),
            pl.BlockSpec((blk, 1), lambda i: (i, 0)),
            pl.BlockSpec((blk, D), lambda i: (i, 0)),
            pl.BlockSpec((1, D), lambda i: (0, 0)),
        ],
        out_specs=pl.BlockSpec((blk, D), lambda i: (i, 0)),
        out_shape=jax.ShapeDtypeStruct((N, D), jnp.float32),
    )(acc, acc, dis[:N].reshape(N, 1), h, b.reshape(1, D))


def _sc_body(pk_hbm, ew_hbm, h_hbm, out_hbm, dis_hbm,
             pk_v, ew_v, norm_v, rows_v, dis_full_v, deg_loc_v,
             acc_s, deg_s, dis_s, sem_ld, sem_g, sem_st):
    cid = lax.axis_index("c")
    sid = lax.axis_index("s")

    zero16 = jnp.zeros((16,), jnp.float32)
    two16 = jnp.full((16,), 2.0, jnp.float32)

    def issue_ld(ch, s):
        pltpu.async_copy(pk_hbm.at[ch], pk_v.at[s], sem_ld.at[s])
        pltpu.async_copy(ew_hbm.at[pl.ds(ch * C, C)], ew_v.at[s],
                         sem_ld.at[s])

    def wait_ld(ch, s):
        pltpu.make_async_copy(pk_hbm.at[ch], pk_v.at[s], sem_ld.at[s]).wait()
        pltpu.make_async_copy(ew_hbm.at[pl.ds(ch * C, C)], ew_v.at[s],
                              sem_ld.at[s]).wait()

    @pl.loop(0, C)
    def _(i):
        for j in range(D // 16):
            rows_v[0, i, pl.ds(j * 16, 16)] = zero16

    @pl.loop(0, RPT // 16)
    def _(k):
        deg_loc_v[pl.ds(k * 16, 16)] = two16

    pltpu.sync_copy(deg_loc_v, deg_s.at[pl.ds(sid * RPT, RPT)])

    @pl.loop(0, RPT // C)
    def _(k):
        pltpu.sync_copy(rows_v.at[0], acc_s.at[pl.ds(sid * RPT + k * C, C)])

    plsc.subcore_barrier()

    dbase = sid * NCH_D
    for s in range(NSLOT):
        issue_ld(dbase + s, s)

    @pl.loop(0, (NCH_D + 3) // NSLOT + 1)
    def _(o):
        for s in range(NSLOT):
            k = o * NSLOT + s

            @pl.when(k < NCH_D)
            def _():
                wait_ld(dbase + k, s)
                pltpu.async_copy(ew_v.at[s], deg_s.at[pk_v.at[s, 1]],
                                 sem_st.at[s], add=True)

            @pl.when(jnp.logical_and(k >= 2, k + 2 < NCH_D))
            def _():
                s2 = (s + 2) % NSLOT
                pltpu.make_async_copy(ew_v.at[s2],
                                      deg_s.at[pk_v.at[s2, 1]],
                                      sem_st.at[s2]).wait()
                issue_ld(dbase + k + 2, s2)

    for s in range(NSLOT):
        pltpu.make_async_copy(ew_v.at[s], deg_s.at[pk_v.at[s, 1]],
                              sem_st.at[s]).wait()

    plsc.subcore_barrier()

    base = sid * RPT
    pltpu.sync_copy(deg_s.at[pl.ds(base, RPT)], deg_loc_v)

    @pl.loop(0, RPT // 16)
    def _(k):
        xv = deg_loc_v[pl.ds(k * 16, 16)]
        ii = lax.bitcast_convert_type(xv, jnp.int32)
        ii = jnp.int32(0x5F3759DF) - lax.shift_right_logical(ii, 1)
        y = lax.bitcast_convert_type(ii, jnp.float32)
        for _ in range(3):
            y = y * (1.5 - 0.5 * xv * y * y)
        deg_loc_v[pl.ds(k * 16, 16)] = y

    pltpu.sync_copy(deg_loc_v, dis_s.at[pl.ds(base, RPT)])

    @pl.when(cid == 0)
    def _():
        pltpu.sync_copy(deg_loc_v, dis_hbm.at[pl.ds(base, RPT)])

    plsc.subcore_barrier()
    pltpu.sync_copy(dis_s, dis_full_v)

    nch_m = jnp.where(cid == 0, NCH_M0, NCH_M1)
    mbase = (cid * NS * NCH_M0
             + sid * nch_m)
    for s in range(NSLOT):
        issue_ld(mbase + s, s)

    @pl.loop(0, (max(NCH_M0, NCH_M1) + 1 + 3) // NSLOT + 1)
    def _(o):
        for s in range(NSLOT):
            k = o * NSLOT + s
            p = (s + 3) % NSLOT
            s2 = (s + 2) % NSLOT

            @pl.when(k < nch_m)
            def _():
                wait_ld(mbase + k, s)
                pltpu.async_copy(h_hbm.at[pk_v.at[s, 0]], rows_v.at[s],
                                 sem_g.at[s])

            @pl.when(jnp.logical_and(k >= 1, k - 1 < nch_m))
            def _():
                pltpu.make_async_copy(h_hbm.at[pk_v.at[p, 0]], rows_v.at[p],
                                      sem_g.at[p]).wait()
                for j in range(C // 16):
                    ir = pk_v[p, 0, pl.ds(j * 16, 16)]
                    ic = pk_v[p, 1, pl.ds(j * 16, 16)]
                    dr = plsc.load_gather(dis_full_v, [ir])
                    dc = plsc.load_gather(dis_full_v, [ic])
                    norm_v[pl.ds(j * 16, 16)] = (
                        dr * ew_v[p, pl.ds(j * 16, 16)] * dc)

                @pl.loop(0, C)
                def _(i):
                    nb = plsc.load_gather(norm_v,
                                          [jnp.full((16,), i, jnp.int32)])
                    for j in range(D // 16):
                        rows_v[p, i, pl.ds(j * 16, 16)] = (
                            rows_v[p, i, pl.ds(j * 16, 16)] * nb)

                pltpu.async_copy(rows_v.at[p], acc_s.at[pk_v.at[p, 1]],
                                 sem_st.at[p], add=True)

            @pl.when(jnp.logical_and(k >= 2, k + 2 < nch_m))
            def _():
                pltpu.make_async_copy(rows_v.at[s2],
                                      acc_s.at[pk_v.at[s2, 1]],
                                      sem_st.at[s2]).wait()
                issue_ld(mbase + k + 2, s2)

    for s in range(NSLOT):
        pltpu.make_async_copy(rows_v.at[s], acc_s.at[pk_v.at[s, 1]],
                              sem_st.at[s]).wait()

    plsc.subcore_barrier()

    wb = sid * RPT
    pltpu.sync_copy(acc_s.at[pl.ds(wb, RPT)], out_hbm.at[cid, pl.ds(wb, RPT)])


_sc_agg = functools.partial(
    pl.kernel,
    out_type=(
        jax.ShapeDtypeStruct((NC, NP, D), jnp.float32),
        jax.ShapeDtypeStruct((NP,), jnp.float32),
    ),
    mesh=plsc.VectorSubcoreMesh(core_axis_name="c", subcore_axis_name="s"),
    compiler_params=pltpu.CompilerParams(needs_layout_passes=False),
    scratch_types=(
        pltpu.VMEM((NSLOT, 2, C), jnp.int32),
        pltpu.VMEM((NSLOT, C), jnp.float32),
        pltpu.VMEM((C,), jnp.float32),
        pltpu.VMEM((NSLOT, C, D), jnp.float32),
        pltpu.VMEM((NP,), jnp.float32),
        pltpu.VMEM((RPT,), jnp.float32),
        pltpu.VMEM_SHARED((NP, D), jnp.float32),
        pltpu.VMEM_SHARED((NP,), jnp.float32),
        pltpu.VMEM_SHARED((NP,), jnp.float32),
        pltpu.SemaphoreType.DMA((NSLOT,)),
        pltpu.SemaphoreType.DMA((NSLOT,)),
        pltpu.SemaphoreType.DMA((NSLOT,)),
    ),
)(_sc_body)


def kernel(x, edge_index, edge_weight, W, b):
    row = edge_index[0].astype(jnp.int32)
    col = edge_index[1].astype(jnp.int32)
    ew = edge_weight.astype(jnp.float32)
    pad = E_PAD - E
    row_p = jnp.concatenate([row, jnp.zeros((pad,), jnp.int32)])
    col_p = jnp.concatenate([col, jnp.zeros((pad,), jnp.int32)])
    ew_p = jnp.concatenate([ew, jnp.zeros((pad,), jnp.float32)])
    pk = jnp.stack([row_p.reshape(NCHUNK, C), col_p.reshape(NCHUNK, C)],
                   axis=1)
    h = _matmul(x, W)
    acc, dis = _sc_agg(pk, ew_p, h)
    return _combine(acc, dis, h, b)


# speedup vs baseline: 1.1470x; 1.0016x over previous
import functools

import jax
import jax.numpy as jnp
from jax import lax
from jax.experimental import pallas as pl
from jax.experimental.pallas import tpu as pltpu
from jax.experimental.pallas import tpu_sc as plsc

N = 10000
E = 320000
D = 128

NC = 2
NS = 16
NP = 10240
RPT = NP // NS
C = 64
E_PAD = 321536
NCHUNK = E_PAD // C
NCH_PAIR = NCHUNK // NS
NCH_M0 = NCH_PAIR // 2
NCH_M1 = NCH_PAIR - NCH_M0
NCH_D = NCHUNK // NS
NSLOT = 4


def _matmul_body(x_ref, w_ref, o_ref):
    o_ref[...] = jnp.dot(x_ref[...], w_ref[...],
                         preferred_element_type=jnp.float32)


def _matmul(x, W):
    return pl.pallas_call(
        _matmul_body,
        out_shape=jax.ShapeDtypeStruct((N, D), jnp.float32),
    )(x, W)


def _combine_body(a0_ref, a1_ref, d_ref, h_ref, b_ref, o_ref):
    d = d_ref[...]
    o_ref[...] = jnp.maximum(
        a0_ref[0] + a1_ref[0] + (2.0 * d * d) * h_ref[...] + b_ref[...],
        0.0)


def _combine(acc, dis, h, b):
    blk = 2000
    return pl.pallas_call(
        _combine_body,
        grid=(N // blk,),
        in_specs=[
            pl.BlockSpec((1, blk, D), lambda i: (0, i, 0)),
            pl.BlockSpec((1, blk, D), lambda i: (1, i, 0)),
            pl.BlockSpec((blk, 1), lambda i: (i, 0)),
            pl.BlockSpec((blk, D), lambda i: (i, 0)),
            pl.BlockSpec((1, D), lambda i: (0, 0)),
        ],
        out_specs=pl.BlockSpec((blk, D), lambda i: (i, 0)),
        out_shape=jax.ShapeDtypeStruct((N, D), jnp.float32),
    )(acc, acc, dis[:N].reshape(N, 1), h, b.reshape(1, D))


def _sc_body(pk_hbm, ew_hbm, h_hbm, out_hbm, dis_hbm,
             pk_v, ew_v, norm_v, rows_v, dis_full_v, deg_loc_v,
             acc_s, deg_s, dis_s, sem_ld, sem_g, sem_st):
    cid = lax.axis_index("c")
    sid = lax.axis_index("s")

    zero16 = jnp.zeros((16,), jnp.float32)
    two16 = jnp.full((16,), 2.0, jnp.float32)

    def issue_ld(ch, s):
        pltpu.async_copy(pk_hbm.at[ch], pk_v.at[s], sem_ld.at[s])
        pltpu.async_copy(ew_hbm.at[pl.ds(ch * C, C)], ew_v.at[s],
                         sem_ld.at[s])

    def wait_ld(ch, s):
        pltpu.make_async_copy(pk_hbm.at[ch], pk_v.at[s], sem_ld.at[s]).wait()
        pltpu.make_async_copy(ew_hbm.at[pl.ds(ch * C, C)], ew_v.at[s],
                              sem_ld.at[s]).wait()

    @pl.loop(0, C)
    def _(i):
        for j in range(D // 16):
            rows_v[0, i, pl.ds(j * 16, 16)] = zero16

    @pl.loop(0, RPT // 16)
    def _(k):
        deg_loc_v[pl.ds(k * 16, 16)] = two16

    pltpu.sync_copy(deg_loc_v, deg_s.at[pl.ds(sid * RPT, RPT)])

    for k in range(RPT // C):
        pltpu.async_copy(rows_v.at[0], acc_s.at[pl.ds(sid * RPT + k * C, C)],
                         sem_g.at[0])
    for k in range(RPT // C):
        pltpu.make_async_copy(rows_v.at[0],
                              acc_s.at[pl.ds(sid * RPT, C)],
                              sem_g.at[0]).wait()

    plsc.subcore_barrier()

    dbase = sid * NCH_D
    for s in range(NSLOT):
        issue_ld(dbase + s, s)

    @pl.loop(0, (NCH_D + 3) // NSLOT + 1)
    def _(o):
        for s in range(NSLOT):
            k = o * NSLOT + s

            @pl.when(k < NCH_D)
            def _():
                wait_ld(dbase + k, s)
                pltpu.async_copy(ew_v.at[s], deg_s.at[pk_v.at[s, 1]],
                                 sem_st.at[s], add=True)

            @pl.when(jnp.logical_and(k >= 2, k + 2 < NCH_D))
            def _():
                s2 = (s + 2) % NSLOT
                pltpu.make_async_copy(ew_v.at[s2],
                                      deg_s.at[pk_v.at[s2, 1]],
                                      sem_st.at[s2]).wait()
                issue_ld(dbase + k + 2, s2)

    for s in range(NSLOT):
        pltpu.make_async_copy(ew_v.at[s], deg_s.at[pk_v.at[s, 1]],
                              sem_st.at[s]).wait()

    plsc.subcore_barrier()

    base = sid * RPT
    pltpu.sync_copy(deg_s.at[pl.ds(base, RPT)], deg_loc_v)

    @pl.loop(0, RPT // 16)
    def _(k):
        xv = deg_loc_v[pl.ds(k * 16, 16)]
        ii = lax.bitcast_convert_type(xv, jnp.int32)
        ii = jnp.int32(0x5F3759DF) - lax.shift_right_logical(ii, 1)
        y = lax.bitcast_convert_type(ii, jnp.float32)
        for _ in range(3):
            y = y * (1.5 - 0.5 * xv * y * y)
        deg_loc_v[pl.ds(k * 16, 16)] = y

    pltpu.sync_copy(deg_loc_v, dis_s.at[pl.ds(base, RPT)])

    @pl.when(cid == 0)
    def _():
        pltpu.sync_copy(deg_loc_v, dis_hbm.at[pl.ds(base, RPT)])

    plsc.subcore_barrier()
    pltpu.sync_copy(dis_s, dis_full_v)

    nch_m = jnp.where(cid == 0, NCH_M0, NCH_M1)
    mbase = (cid * NS * NCH_M0
             + sid * nch_m)
    for s in range(NSLOT):
        issue_ld(mbase + s, s)

    @pl.loop(0, (max(NCH_M0, NCH_M1) + 1 + 3) // NSLOT + 1)
    def _(o):
        for s in range(NSLOT):
            k = o * NSLOT + s
            p = (s + 3) % NSLOT
            s2 = (s + 2) % NSLOT

            @pl.when(k < nch_m)
            def _():
                wait_ld(mbase + k, s)
                pltpu.async_copy(h_hbm.at[pk_v.at[s, 0]], rows_v.at[s],
                                 sem_g.at[s])

            @pl.when(jnp.logical_and(k >= 1, k - 1 < nch_m))
            def _():
                pltpu.make_async_copy(h_hbm.at[pk_v.at[p, 0]], rows_v.at[p],
                                      sem_g.at[p]).wait()
                for j in range(C // 16):
                    ir = pk_v[p, 0, pl.ds(j * 16, 16)]
                    ic = pk_v[p, 1, pl.ds(j * 16, 16)]
                    dr = plsc.load_gather(dis_full_v, [ir])
                    dc = plsc.load_gather(dis_full_v, [ic])
                    norm_v[pl.ds(j * 16, 16)] = (
                        dr * ew_v[p, pl.ds(j * 16, 16)] * dc)

                @pl.loop(0, C)
                def _(i):
                    nb = plsc.load_gather(norm_v,
                                          [jnp.full((16,), i, jnp.int32)])
                    for j in range(D // 16):
                        rows_v[p, i, pl.ds(j * 16, 16)] = (
                            rows_v[p, i, pl.ds(j * 16, 16)] * nb)

                pltpu.async_copy(rows_v.at[p], acc_s.at[pk_v.at[p, 1]],
                                 sem_st.at[p], add=True)

            @pl.when(jnp.logical_and(k >= 2, k + 2 < nch_m))
            def _():
                pltpu.make_async_copy(rows_v.at[s2],
                                      acc_s.at[pk_v.at[s2, 1]],
                                      sem_st.at[s2]).wait()
                issue_ld(mbase + k + 2, s2)

    for s in range(NSLOT):
        pltpu.make_async_copy(rows_v.at[s], acc_s.at[pk_v.at[s, 1]],
                              sem_st.at[s]).wait()

    plsc.subcore_barrier()

    wb = sid * RPT
    pltpu.sync_copy(acc_s.at[pl.ds(wb, RPT)], out_hbm.at[cid, pl.ds(wb, RPT)])


_sc_agg = functools.partial(
    pl.kernel,
    out_type=(
        jax.ShapeDtypeStruct((NC, NP, D), jnp.float32),
        jax.ShapeDtypeStruct((NP,), jnp.float32),
    ),
    mesh=plsc.VectorSubcoreMesh(core_axis_name="c", subcore_axis_name="s"),
    compiler_params=pltpu.CompilerParams(needs_layout_passes=False),
    scratch_types=(
        pltpu.VMEM((NSLOT, 2, C), jnp.int32),
        pltpu.VMEM((NSLOT, C), jnp.float32),
        pltpu.VMEM((C,), jnp.float32),
        pltpu.VMEM((NSLOT, C, D), jnp.float32),
        pltpu.VMEM((NP,), jnp.float32),
        pltpu.VMEM((RPT,), jnp.float32),
        pltpu.VMEM_SHARED((NP, D), jnp.float32),
        pltpu.VMEM_SHARED((NP,), jnp.float32),
        pltpu.VMEM_SHARED((NP,), jnp.float32),
        pltpu.SemaphoreType.DMA((NSLOT,)),
        pltpu.SemaphoreType.DMA((NSLOT,)),
        pltpu.SemaphoreType.DMA((NSLOT,)),
    ),
)(_sc_body)


def kernel(x, edge_index, edge_weight, W, b):
    row = edge_index[0].astype(jnp.int32)
    col = edge_index[1].astype(jnp.int32)
    ew = edge_weight.astype(jnp.float32)
    pad = E_PAD - E
    row_p = jnp.concatenate([row, jnp.zeros((pad,), jnp.int32)])
    col_p = jnp.concatenate([col, jnp.zeros((pad,), jnp.int32)])
    ew_p = jnp.concatenate([ew, jnp.zeros((pad,), jnp.float32)])
    pk = jnp.stack([row_p.reshape(NCHUNK, C), col_p.reshape(NCHUNK, C)],
                   axis=1)
    h = _matmul(x, W)
    acc, dis = _sc_agg(pk, ew_p, h)
    return _combine(acc, dis, h, b)
